# baseline (speedup 1.0000x reference)
import functools

import jax
import jax.numpy as jnp
from jax import lax
from jax.experimental import pallas as pl
from jax.experimental.pallas import tpu as pltpu
from jax.experimental.pallas import tpu_sc as plsc

DIM = 128
SLOPE = 0.1
NW = 32
LANES = 16
EB = 128


def _dense_body(f_ref, w_ref, b_ref, at_ref, ab_ref, ne_ref, sc_ref):
    ne = jnp.dot(f_ref[...], w_ref[...], preferred_element_type=jnp.float32)
    ne = ne + b_ref[...]
    ne_ref[...] = ne
    sc_ref[0, :] = jnp.sum(ne * at_ref[...], axis=1)
    sc_ref[1, :] = jnp.sum(ne * ab_ref[...], axis=1)


def _combine_body(a0_ref, a1_ref, rs_ref, o_ref):
    tot = a0_ref[...] + a1_ref[...]
    r = jnp.sum(rs_ref[...], axis=0)
    o_ref[...] = tot / r[:, None]


def _make_sc_kernel(npad, steps):
    rows_per_tile = npad // LANES
    mesh = plsc.VectorSubcoreMesh(core_axis_name="c", subcore_axis_name="s")

    @functools.partial(
        pl.kernel,
        out_type=(
            jax.ShapeDtypeStruct((2, npad, DIM), jnp.float32),
            jax.ShapeDtypeStruct((2, npad), jnp.float32),
        ),
        mesh=mesh,
        scratch_types=[
            [pltpu.VMEM((EB,), jnp.int32)] * 2,
            [pltpu.VMEM((EB,), jnp.int32)] * 2,
            [pltpu.VMEM((EB, DIM), jnp.float32)] * 2,
            [pltpu.VMEM((EB,), jnp.float32)] * 2,
            [pltpu.VMEM((EB,), jnp.float32)] * 2,
            [pltpu.VMEM((EB,), jnp.float32)] * 2,
            pltpu.VMEM((npad // LANES,), jnp.float32),
            pltpu.VMEM_SHARED((npad, DIM), jnp.float32),
            pltpu.VMEM_SHARED((npad,), jnp.float32),
            [pltpu.SemaphoreType.DMA] * 2,
            [pltpu.SemaphoreType.DMA] * 2,
        ],
        compiler_params=pltpu.CompilerParams(needs_layout_passes=False),
    )
    def sc_kernel(ne_hbm, s_hbm, d_hbm, src_hbm, dst_hbm,
                  acc_out, rs_out,
                  src_b, dst_b, rows, sv, dv, vals, zeros1d,
                  acc, rs_sh, sem, ssem):
        cid = lax.axis_index("c")
        sid = lax.axis_index("s")
        wid = sid * 2 + cid

        zero16 = jnp.zeros((LANES,), jnp.float32)

        def zrow(j, carry):
            for c8 in range(DIM // LANES):
                rows[0][j, pl.ds(c8 * LANES, LANES)] = zero16
            return carry
        lax.fori_loop(0, EB, zrow, 0)

        def zz(j, carry):
            zeros1d[pl.ds(j * LANES, LANES)] = zero16
            return carry
        lax.fori_loop(0, rows_per_tile // LANES, zz, 0)

        for t in range(rows_per_tile // EB):
            pltpu.sync_copy(rows[0],
                            acc.at[pl.ds(sid * rows_per_tile + t * EB, EB)])
        pltpu.sync_copy(zeros1d, rs_sh.at[pl.ds(sid * rows_per_tile, rows_per_tile)])
        plsc.subcore_barrier()

        def prefetch(st, k):
            pltpu.sync_copy(src_hbm.at[wid, st], src_b[k])
            pltpu.sync_copy(dst_hbm.at[wid, st], dst_b[k])
            pltpu.make_async_copy(ne_hbm.at[dst_b[k]], rows[k], sem[k]).start()
            pltpu.make_async_copy(s_hbm.at[src_b[k]], sv[k], sem[k]).start()
            pltpu.make_async_copy(d_hbm.at[dst_b[k]], dv[k], sem[k]).start()

        def compute(k):
            pltpu.make_async_copy(ne_hbm.at[dst_b[k]], rows[k], sem[k]).wait()
            pltpu.make_async_copy(s_hbm.at[src_b[k]], sv[k], sem[k]).wait()
            pltpu.make_async_copy(d_hbm.at[dst_b[k]], dv[k], sem[k]).wait()

            def grp(gg, carry2):
                sl16 = pl.ds(gg * LANES, LANES)
                x = sv[k][sl16] + dv[k][sl16]
                v = jnp.exp(jnp.maximum(x, x * SLOPE))
                vals[k][sl16] = v
                for l in range(LANES):
                    vv = jnp.broadcast_to(v[l], (LANES,))
                    j = gg * LANES + l
                    for c8 in range(DIM // LANES):
                        sl = pl.ds(c8 * LANES, LANES)
                        rows[k][j, sl] = rows[k][j, sl] * vv
                return carry2
            lax.fori_loop(0, EB // LANES, grp, 0)

        def scatter_start(k):
            pltpu.async_copy(rows[k], acc.at[src_b[k]], ssem[k], add=True)
            pltpu.async_copy(vals[k], rs_sh.at[src_b[k]], ssem[k], add=True)

        def scatter_wait(k):
            pltpu.make_async_copy(rows[k], acc.at[src_b[k]], ssem[k]).wait()
            pltpu.make_async_copy(vals[k], rs_sh.at[src_b[k]], ssem[k]).wait()

        prefetch(0, 0)

        def body(i, carry):
            st = i * 2

            @pl.when(st > 0)
            def _():
                scatter_wait(1)
            prefetch(st + 1, 1)
            compute(0)
            scatter_start(0)
            compute(1)
            @pl.when(st + 2 < steps)
            def _():
                scatter_wait(0)
                prefetch(st + 2, 0)
            scatter_start(1)
            return carry
        lax.fori_loop(0, steps // 2, body, 0)

        scatter_wait(0)
        scatter_wait(1)
        plsc.subcore_barrier()
        pltpu.sync_copy(acc.at[pl.ds(sid * rows_per_tile, rows_per_tile)],
                        acc_out.at[cid, pl.ds(sid * rows_per_tile, rows_per_tile)])
        pltpu.sync_copy(rs_sh.at[pl.ds(sid * rows_per_tile, rows_per_tile)],
                        rs_out.at[cid, pl.ds(sid * rows_per_tile, rows_per_tile)])

    return sc_kernel


def kernel(features, W, b, a, nodes, edge_index, ind):
    n = features.shape[0]
    npad = -((n + 1) // -(LANES * EB)) * (LANES * EB)

    n_edges = edge_index.shape[1] + nodes.shape[0]
    steps = 2 * -(n_edges // -(NW * EB * 2))
    epad = NW * steps * EB

    feat_pad = jnp.pad(features, ((0, npad - n), (0, 0)))
    a_top = a[:DIM, 0].reshape(1, DIM)
    a_bot = a[DIM:, 0].reshape(1, DIM)
    blk = 1024
    ne, scores = pl.pallas_call(
        _dense_body,
        grid=(npad // blk,),
        in_specs=[
            pl.BlockSpec((blk, DIM), lambda i: (i, 0)),
            pl.BlockSpec((DIM, DIM), lambda i: (0, 0)),
            pl.BlockSpec((1, DIM), lambda i: (0, 0)),
            pl.BlockSpec((1, DIM), lambda i: (0, 0)),
            pl.BlockSpec((1, DIM), lambda i: (0, 0)),
        ],
        out_specs=[
            pl.BlockSpec((blk, DIM), lambda i: (i, 0)),
            pl.BlockSpec((2, blk), lambda i: (0, i)),
        ],
        out_shape=[
            jax.ShapeDtypeStruct((npad, DIM), jnp.float32),
            jax.ShapeDtypeStruct((2, npad), jnp.float32),
        ],
    )(feat_pad, W, b.reshape(1, DIM), a_top, a_bot)

    pad_e = epad - n_edges
    pad_src = n + jnp.arange(pad_e, dtype=jnp.int32) % (npad - n)
    src = jnp.concatenate(
        [edge_index[0], nodes, pad_src]).astype(jnp.int32)
    dst = jnp.concatenate(
        [edge_index[1], nodes, jnp.zeros((pad_e,), jnp.int32)]).astype(jnp.int32)
    srcm = src.reshape(NW, steps, EB)
    dstm = dst.reshape(NW, steps, EB)

    acc, rs = _make_sc_kernel(npad, steps)(ne, scores[0], scores[1], srcm, dstm)

    out_pad = pl.pallas_call(
        _combine_body,
        grid=(npad // blk,),
        in_specs=[
            pl.BlockSpec((blk, DIM), lambda i: (i, 0)),
            pl.BlockSpec((blk, DIM), lambda i: (i, 0)),
            pl.BlockSpec((2, blk), lambda i: (0, i)),
        ],
        out_specs=pl.BlockSpec((blk, DIM), lambda i: (i, 0)),
        out_shape=jax.ShapeDtypeStruct((npad, DIM), jnp.float32),
    )(acc[0], acc[1], rs)
    return out_pad[:n]
